# Initial kernel scaffold; baseline (speedup 1.0000x reference)
import jax
import jax.numpy as jnp
from jax.experimental import pallas as pl


def kernel(x, edge_index, W1, a_src1, a_dst1, b1, W2, a_src2, a_dst2, b2, Wc, bc):
    raise NotImplementedError("write your pallas kernel here")


# speedup vs baseline: 45.2600x; 45.2600x over previous
import functools

import jax
import jax.numpy as jnp
import numpy as np
from jax import lax
from jax.experimental import pallas as pl
from jax.experimental.pallas import tpu as pltpu
from jax.experimental.pallas import tpu_sc as plsc

N = 10000
E = 320000
F_IN = 128
HEADS = 8
HID = 16
NCLS = 16
F1 = HEADS * HID

NC = 2
NS = 16
NW = NC * NS
EPW = E // NW
CH = 80
NCHUNK = EPW // CH
STRIPE = N // NS

_f32 = jnp.float32

_mesh = plsc.VectorSubcoreMesh(
    core_axis_name="c", subcore_axis_name="s", num_cores=NC, num_subcores=NS)


def _passA_body(xs_hbm, ad_hbm, src_hbm, dst_hbm, z_hbm, out_hbm,
                src_v, dst_v, xs_v, ad_v, msg_v, acc_sh, sem1, sem2):
    cid = lax.axis_index("c")
    sid = lax.axis_index("s")
    wid = sid * NC + cid
    row0 = sid * STRIPE
    pltpu.sync_copy(z_hbm.at[pl.ds(row0, STRIPE)], acc_sh.at[pl.ds(row0, STRIPE)])
    plsc.subcore_barrier()
    base = wid * EPW

    def chunk(k, carry):
        off = pl.multiple_of(base + k * CH, 8)
        pltpu.sync_copy(src_hbm.at[pl.ds(off, CH)], src_v)
        pltpu.sync_copy(dst_hbm.at[pl.ds(off, CH)], dst_v)
        cp1 = pltpu.async_copy(xs_hbm.at[src_v], xs_v, sem1)
        cp2 = pltpu.async_copy(ad_hbm.at[dst_v], ad_v, sem2)
        cp1.wait()
        cp2.wait()

        def edge(i, c2):
            a_s = xs_v[i, pl.ds(F1, 16)]
            a_d = ad_v[i, pl.ds(0, 16)]
            e = a_s + a_d
            w = jnp.exp(jnp.maximum(e, 0.2 * e))
            msg_v[i, pl.ds(F1, 16)] = w
            for h in range(HEADS):
                msg_v[i, pl.ds(h * HID, HID)] = xs_v[i, pl.ds(h * HID, HID)] * w[h]
            return c2

        lax.fori_loop(0, CH, edge, 0)
        pltpu.sync_copy(msg_v, acc_sh.at[dst_v], add=True)
        return carry

    lax.fori_loop(0, NCHUNK, chunk, 0)
    plsc.subcore_barrier()
    pltpu.sync_copy(acc_sh.at[pl.ds(row0, STRIPE)],
                    out_hbm.at[cid, pl.ds(row0, STRIPE)])


_passA = functools.partial(
    pl.kernel,
    out_type=jax.ShapeDtypeStruct((NC, N, F1 + 16), _f32),
    mesh=_mesh,
    compiler_params=pltpu.CompilerParams(use_tc_tiling_on_sc=False),
    scratch_types=[
        pltpu.VMEM((CH,), jnp.int32),
        pltpu.VMEM((CH,), jnp.int32),
        pltpu.VMEM((CH, F1 + 16), _f32),
        pltpu.VMEM((CH, 16), _f32),
        pltpu.VMEM((CH, F1 + 16), _f32),
        pltpu.VMEM_SHARED((N, F1 + 16), _f32),
        pltpu.SemaphoreType.DMA,
        pltpu.SemaphoreType.DMA,
    ],
)(_passA_body)


def _passCD_body(ts_hbm, td_hbm, src_hbm, dst_hbm, z_hbm, out_hbm,
                 src_v, dst_v, ts_v, td_v, msg_v, acc_sh, sem1, sem2):
    cid = lax.axis_index("c")
    sid = lax.axis_index("s")
    wid = sid * NC + cid
    row0 = sid * STRIPE
    pltpu.sync_copy(z_hbm.at[pl.ds(row0, STRIPE)], acc_sh.at[pl.ds(row0, STRIPE)])
    plsc.subcore_barrier()
    base = wid * EPW

    def chunk(k, carry):
        off = pl.multiple_of(base + k * CH, 8)
        pltpu.sync_copy(src_hbm.at[pl.ds(off, CH)], src_v)
        pltpu.sync_copy(dst_hbm.at[pl.ds(off, CH)], dst_v)
        cp1 = pltpu.async_copy(ts_hbm.at[src_v], ts_v, sem1)
        cp2 = pltpu.async_copy(td_hbm.at[dst_v], td_v, sem2)
        cp1.wait()
        cp2.wait()

        def edge(i, c2):
            vx = ts_v[i, pl.ds(0, 16)]
            va = ts_v[i, pl.ds(16, 16)]
            vd = td_v[i, pl.ds(0, 16)]
            s = va + vd
            w = jnp.exp(jnp.maximum(s, 0.2 * s))
            msg_v[i, pl.ds(0, 16)] = vx * w
            msg_v[i, pl.ds(16, 16)] = w
            return c2

        lax.fori_loop(0, CH, edge, 0)
        pltpu.sync_copy(msg_v, acc_sh.at[dst_v], add=True)
        return carry

    lax.fori_loop(0, NCHUNK, chunk, 0)
    plsc.subcore_barrier()
    pltpu.sync_copy(acc_sh.at[pl.ds(row0, STRIPE)],
                    out_hbm.at[cid, pl.ds(row0, STRIPE)])


_passCD = functools.partial(
    pl.kernel,
    out_type=jax.ShapeDtypeStruct((NC, N, 32), _f32),
    mesh=_mesh,
    compiler_params=pltpu.CompilerParams(use_tc_tiling_on_sc=False),
    scratch_types=[
        pltpu.VMEM((CH,), jnp.int32),
        pltpu.VMEM((CH,), jnp.int32),
        pltpu.VMEM((CH, 32), _f32),
        pltpu.VMEM((CH, 16), _f32),
        pltpu.VMEM((CH, 32), _f32),
        pltpu.VMEM_SHARED((N, 32), _f32),
        pltpu.SemaphoreType.DMA,
        pltpu.SemaphoreType.DMA,
    ],
)(_passCD_body)


def _passE_body(g_hbm, src_hbm, dst_hbm, z_hbm, out_hbm,
                src_v, dst_v, g_v, acc_sh, sem1):
    cid = lax.axis_index("c")
    sid = lax.axis_index("s")
    wid = sid * NC + cid
    row0 = sid * STRIPE
    pltpu.sync_copy(z_hbm.at[pl.ds(row0, STRIPE)], acc_sh.at[pl.ds(row0, STRIPE)])
    plsc.subcore_barrier()
    base = wid * EPW

    def chunk(k, carry):
        off = pl.multiple_of(base + k * CH, 8)
        pltpu.sync_copy(src_hbm.at[pl.ds(off, CH)], src_v)
        pltpu.sync_copy(dst_hbm.at[pl.ds(off, CH)], dst_v)
        pltpu.async_copy(g_hbm.at[src_v], g_v, sem1).wait()
        pltpu.sync_copy(g_v, acc_sh.at[dst_v], add=True)
        return carry

    lax.fori_loop(0, NCHUNK, chunk, 0)
    plsc.subcore_barrier()
    pltpu.sync_copy(acc_sh.at[pl.ds(row0, STRIPE)],
                    out_hbm.at[cid, pl.ds(row0, STRIPE)])


_passE = functools.partial(
    pl.kernel,
    out_type=jax.ShapeDtypeStruct((NC, N, 16), _f32),
    mesh=_mesh,
    compiler_params=pltpu.CompilerParams(use_tc_tiling_on_sc=False),
    scratch_types=[
        pltpu.VMEM((CH,), jnp.int32),
        pltpu.VMEM((CH,), jnp.int32),
        pltpu.VMEM((CH, 16), _f32),
        pltpu.VMEM_SHARED((N, 16), _f32),
        pltpu.SemaphoreType.DMA,
    ],
)(_passE_body)


def _k1_body(x_ref, w1_ref, a1_ref, a2_ref, xs_ref, ad_ref):
    xw = jnp.dot(x_ref[...], w1_ref[...], preferred_element_type=_f32)
    asp = jnp.dot(xw, a1_ref[...], preferred_element_type=_f32)
    xs_ref[...] = jnp.concatenate([xw, asp], axis=1)
    ad_ref[...] = jnp.dot(xw, a2_ref[...], preferred_element_type=_f32)


_k1 = pl.pallas_call(
    _k1_body,
    out_shape=[jax.ShapeDtypeStruct((N, F1 + 16), _f32),
               jax.ShapeDtypeStruct((N, 16), _f32)],
)


def _k2_body(acc_ref, b1_ref, w2_ref, r8_ref, as2w_ref, ad2w_ref,
             ts_ref, td_ref, dinv_ref):
    p = acc_ref[0] + acc_ref[1]
    den = jnp.dot(p[:, F1:F1 + 8], r8_ref[...], preferred_element_type=_f32)
    h1 = p[:, :F1] / (den + 1e-16) + b1_ref[...][None, :]
    h1 = jnp.where(h1 > 0, h1, jnp.exp(jnp.minimum(h1, 0.0)) - 1.0)
    x2 = jnp.dot(h1, w2_ref[...], preferred_element_type=_f32)
    as2 = jnp.dot(x2, as2w_ref[...], preferred_element_type=_f32)
    ts_ref[...] = jnp.concatenate([x2, as2], axis=1)
    td_ref[...] = jnp.dot(x2, ad2w_ref[...], preferred_element_type=_f32)
    deg = p[:, F1 + 8:F1 + 9]
    dinv_ref[...] = lax.rsqrt(1.0 + deg) * jnp.ones((1, 16), _f32)


_k2 = pl.pallas_call(
    _k2_body,
    out_shape=[jax.ShapeDtypeStruct((N, 32), _f32),
               jax.ShapeDtypeStruct((N, 16), _f32),
               jax.ShapeDtypeStruct((N, 16), _f32)],
)


def _k3_body(acc2_ref, b2_ref, wc_ref, dinv_ref, g3_ref, self_ref):
    p = acc2_ref[0] + acc2_ref[1]
    h2 = p[:, :16] / (p[:, 16:17] + 1e-16) + b2_ref[...][None, :]
    h3 = jnp.dot(h2, wc_ref[...], preferred_element_type=_f32)
    dv = dinv_ref[...]
    g3_ref[...] = h3 * dv
    self_ref[...] = h3 * dv * dv


_k3 = pl.pallas_call(
    _k3_body,
    out_shape=[jax.ShapeDtypeStruct((N, 16), _f32),
               jax.ShapeDtypeStruct((N, 16), _f32)],
)


def _k4_body(acc3_ref, dinv_ref, self_ref, bc_ref, out_ref):
    p = acc3_ref[0] + acc3_ref[1]
    out_ref[...] = dinv_ref[...] * p + self_ref[...] + bc_ref[...][None, :]


_k4 = pl.pallas_call(
    _k4_body,
    out_shape=jax.ShapeDtypeStruct((N, 16), _f32),
)


def kernel(x, edge_index, W1, a_src1, a_dst1, b1, W2, a_src2, a_dst2, b2,
           Wc, bc):
    src = edge_index[0].astype(jnp.int32)
    dst = edge_index[1].astype(jnp.int32)
    eye8 = jnp.eye(HEADS, dtype=_f32)
    A1 = jnp.pad((a_src1[:, :, None] * eye8[:, None, :]).reshape(F1, HEADS),
                 ((0, 0), (0, 8)))
    A2 = jnp.pad((a_dst1[:, :, None] * eye8[:, None, :]).reshape(F1, HEADS),
                 ((0, 0), (0, 8)))
    R8 = jnp.asarray(np.kron(np.eye(8, dtype=np.float32),
                             np.ones((1, 16), np.float32)))
    AS2W = a_src2.reshape(NCLS, 1) * jnp.ones((1, 16), _f32)
    AD2W = a_dst2.reshape(NCLS, 1) * jnp.ones((1, 16), _f32)

    XS, AD = _k1(x, W1, A1, A2)
    Z144 = jnp.zeros((N, F1 + 16), _f32)
    ACC1 = _passA(XS, AD, src, dst, Z144)
    TS2, TD2, DINV = _k2(ACC1, b1, W2, R8, AS2W, AD2W)
    Z32 = jnp.zeros((N, 32), _f32)
    ACC2 = _passCD(TS2, TD2, src, dst, Z32)
    G3, SELF = _k3(ACC2, b2, Wc, DINV)
    Z16 = jnp.zeros((N, 16), _f32)
    ACC3 = _passE(G3, src, dst, Z16)
    return _k4(ACC3, DINV, SELF, bc)


# speedup vs baseline: 68.3184x; 1.5095x over previous
import functools

import jax
import jax.numpy as jnp
import numpy as np
from jax import lax
from jax.experimental import pallas as pl
from jax.experimental.pallas import tpu as pltpu
from jax.experimental.pallas import tpu_sc as plsc

N = 10000
E = 320000
F_IN = 128
HEADS = 8
HID = 16
NCLS = 16
F1 = HEADS * HID

NC = 2
NS = 16
NW = NC * NS
N_PAD = 10016
E_PAD = 322560
EPW = E_PAD // NW
STRIPE = N_PAD // NS

_f32 = jnp.float32

_mesh = plsc.VectorSubcoreMesh(
    core_axis_name="c", subcore_axis_name="s", num_cores=NC, num_subcores=NS)


def _build_edge_pass(widths, idx_kinds, msg_w, compute_fn, ch):
    n_t = len(widths)
    nchunk = EPW // ch
    assert EPW % ch == 0 and nchunk % 2 == 0 and ch % 16 == 0 and ch <= 128

    def body(*refs):
        t_hbm = refs[:n_t]
        src_hbm, dst_hbm, z_hbm, out_hbm = refs[n_t:n_t + 4]
        s = list(refs[n_t + 4:])
        srcg = s[0:2]
        dstg = s[2:4]
        dsts = s[4:6]
        gb = [s[6 + 2 * t:8 + 2 * t] for t in range(n_t)]
        o = 6 + 2 * n_t
        msg = s[o:o + 2]
        acc = s[o + 2]
        sem_ix = s[o + 3:o + 5]
        sem_g = [s[o + 5 + 2 * t:o + 7 + 2 * t] for t in range(n_t)]
        sem_sc = s[o + 5 + 2 * n_t:o + 7 + 2 * n_t]
        gidx = [srcg if kind == "src" else dstg for kind in idx_kinds]

        cid = lax.axis_index("c")
        sid = lax.axis_index("s")
        wid = sid * NC + cid
        row0 = sid * STRIPE
        base = wid * EPW
        pltpu.sync_copy(z_hbm.at[pl.ds(row0, STRIPE)],
                        acc.at[pl.ds(row0, STRIPE)])
        plsc.subcore_barrier()

        def idx_slice(off):
            return pl.ds(pl.multiple_of(off, 8), ch)

        pltpu.sync_copy(src_hbm.at[idx_slice(base)], srcg[0])
        pltpu.sync_copy(dst_hbm.at[idx_slice(base)], dstg[0])
        for t in range(n_t):
            pltpu.async_copy(t_hbm[t].at[gidx[t][0]], gb[t][0], sem_g[t][0])
        pltpu.async_copy(src_hbm.at[idx_slice(base + ch)], srcg[1], sem_ix[1])
        pltpu.async_copy(dst_hbm.at[idx_slice(base + ch)], dstg[1], sem_ix[1])

        def drain_scatter(b):
            pltpu.make_async_copy(msg[b], acc.at[dsts[b]], sem_sc[b]).wait()

        def half(k, b, nb, steady):
            if steady:
                pltpu.make_async_copy(src_hbm.at[idx_slice(base)], srcg[nb],
                                      sem_ix[nb]).wait()
                pltpu.make_async_copy(dst_hbm.at[idx_slice(base)], dstg[nb],
                                      sem_ix[nb]).wait()
                for t in range(n_t):
                    pltpu.async_copy(t_hbm[t].at[gidx[t][nb]], gb[t][nb],
                                     sem_g[t][nb])

                @pl.when(k >= 2)
                def _():
                    drain_scatter(b)
            else:
                drain_scatter(b)
            for t in range(n_t):
                pltpu.make_async_copy(t_hbm[t].at[gidx[t][b]], gb[t][b],
                                      sem_g[t][b]).wait()
            for q in range(ch // 16):
                dsts[b][pl.ds(16 * q, 16)] = dstg[b][pl.ds(16 * q, 16)]

            def edge(i, c2):
                compute_fn(i, [gb[t][b] for t in range(n_t)], msg[b])
                return c2

            lax.fori_loop(0, ch, edge, 0)
            pltpu.async_copy(msg[b], acc.at[dsts[b]], sem_sc[b], add=True)
            if steady:
                @pl.when(k + 2 <= nchunk - 1)
                def _():
                    off2 = base + (k + 2) * ch
                    pltpu.async_copy(src_hbm.at[idx_slice(off2)], srcg[b],
                                     sem_ix[b])
                    pltpu.async_copy(dst_hbm.at[idx_slice(off2)], dstg[b],
                                     sem_ix[b])

        def loop_body(j, c):
            half(2 * j, 0, 1, True)
            half(2 * j + 1, 1, 0, True)
            return c

        lax.fori_loop(0, (nchunk - 2) // 2, loop_body, 0)
        half(nchunk - 2, 0, 1, True)
        half(nchunk - 1, 1, 0, False)
        drain_scatter(0)
        drain_scatter(1)
        plsc.subcore_barrier()
        pltpu.sync_copy(acc.at[pl.ds(row0, STRIPE)],
                        out_hbm.at[cid, pl.ds(row0, STRIPE)])

    return body


def _edge_pass(widths, idx_kinds, msg_w, compute_fn, ch):
    st = [pltpu.VMEM((ch,), jnp.int32) for _ in range(6)]
    for w in widths:
        st += [pltpu.VMEM((ch, w), _f32) for _ in range(2)]
    st += [pltpu.VMEM((ch, msg_w), _f32) for _ in range(2)]
    st += [pltpu.VMEM_SHARED((N_PAD, msg_w), _f32)]
    st += [pltpu.SemaphoreType.DMA for _ in range(4 + 2 * len(widths))]
    return functools.partial(
        pl.kernel,
        out_type=jax.ShapeDtypeStruct((NC, N_PAD, msg_w), _f32),
        mesh=_mesh,
        compiler_params=pltpu.CompilerParams(use_tc_tiling_on_sc=False),
        scratch_types=st,
    )(_build_edge_pass(widths, idx_kinds, msg_w, compute_fn, ch))


def _passA_compute(i, gb, msg):
    xs_v, ad_v = gb
    a_s = xs_v[i, pl.ds(F1, 16)]
    a_d = ad_v[i, pl.ds(0, 16)]
    e = a_s + a_d
    w = jnp.exp(jnp.maximum(e, 0.2 * e))
    msg[i, pl.ds(F1, 16)] = w
    for h in range(HEADS):
        msg[i, pl.ds(h * HID, HID)] = xs_v[i, pl.ds(h * HID, HID)] * w[h]


_passA = _edge_pass([F1 + 16, 16], ["src", "dst"], F1 + 16, _passA_compute,
                    ch=48)


def _passCD_compute(i, gb, msg):
    ts_v, td_v = gb
    vx = ts_v[i, pl.ds(0, 16)]
    va = ts_v[i, pl.ds(16, 16)]
    vd = td_v[i, pl.ds(0, 16)]
    s = va + vd
    w = jnp.exp(jnp.maximum(s, 0.2 * s))
    msg[i, pl.ds(0, 16)] = vx * w
    msg[i, pl.ds(16, 16)] = w


_passCD = _edge_pass([32, 16], ["src", "dst"], 32, _passCD_compute, ch=112)


def _passE_compute(i, gb, msg):
    msg[i, pl.ds(0, 16)] = gb[0][i, pl.ds(0, 16)]


_passE = _edge_pass([16], ["src"], 16, _passE_compute, ch=112)


def _k1_body(x_ref, w1_ref, a1_ref, a2_ref, xs_ref, ad_ref):
    xw = jnp.dot(x_ref[...], w1_ref[...], preferred_element_type=_f32)
    asp = jnp.dot(xw, a1_ref[...], preferred_element_type=_f32)
    xs_ref[...] = jnp.concatenate([xw, asp], axis=1)
    ad_ref[...] = jnp.dot(xw, a2_ref[...], preferred_element_type=_f32)


_k1 = pl.pallas_call(
    _k1_body,
    out_shape=[jax.ShapeDtypeStruct((N_PAD, F1 + 16), _f32),
               jax.ShapeDtypeStruct((N_PAD, 16), _f32)],
)


def _k2_body(acc_ref, b1_ref, w2_ref, r8_ref, as2w_ref, ad2w_ref,
             ts_ref, td_ref, dinv_ref):
    p = acc_ref[0] + acc_ref[1]
    den = jnp.dot(p[:, F1:F1 + 8], r8_ref[...], preferred_element_type=_f32)
    h1 = p[:, :F1] / (den + 1e-16) + b1_ref[...][None, :]
    h1 = jnp.where(h1 > 0, h1, jnp.exp(jnp.minimum(h1, 0.0)) - 1.0)
    x2 = jnp.dot(h1, w2_ref[...], preferred_element_type=_f32)
    as2 = jnp.dot(x2, as2w_ref[...], preferred_element_type=_f32)
    ts_ref[...] = jnp.concatenate([x2, as2], axis=1)
    td_ref[...] = jnp.dot(x2, ad2w_ref[...], preferred_element_type=_f32)
    deg = p[:, F1 + 8:F1 + 9]
    dinv_ref[...] = lax.rsqrt(1.0 + deg) * jnp.ones((1, 16), _f32)


_k2 = pl.pallas_call(
    _k2_body,
    out_shape=[jax.ShapeDtypeStruct((N_PAD, 32), _f32),
               jax.ShapeDtypeStruct((N_PAD, 16), _f32),
               jax.ShapeDtypeStruct((N_PAD, 16), _f32)],
)


def _k3_body(acc2_ref, b2_ref, wc_ref, dinv_ref, g3_ref, self_ref):
    p = acc2_ref[0] + acc2_ref[1]
    h2 = p[:, :16] / (p[:, 16:17] + 1e-16) + b2_ref[...][None, :]
    h3 = jnp.dot(h2, wc_ref[...], preferred_element_type=_f32)
    dv = dinv_ref[...]
    g3_ref[...] = h3 * dv
    self_ref[...] = h3 * dv * dv


_k3 = pl.pallas_call(
    _k3_body,
    out_shape=[jax.ShapeDtypeStruct((N_PAD, 16), _f32),
               jax.ShapeDtypeStruct((N_PAD, 16), _f32)],
)


def _k4_body(acc3_ref, dinv_ref, self_ref, bc_ref, out_ref):
    p = acc3_ref[0] + acc3_ref[1]
    out_ref[...] = dinv_ref[...] * p + self_ref[...] + bc_ref[...][None, :]


_k4 = pl.pallas_call(
    _k4_body,
    out_shape=jax.ShapeDtypeStruct((N_PAD, 16), _f32),
)


def kernel(x, edge_index, W1, a_src1, a_dst1, b1, W2, a_src2, a_dst2, b2,
           Wc, bc):
    src = edge_index[0].astype(jnp.int32)
    dst = edge_index[1].astype(jnp.int32)
    pad_e = E_PAD - E
    src = jnp.concatenate([src, jnp.full((pad_e,), N, jnp.int32)])
    dst = jnp.concatenate([dst, jnp.full((pad_e,), N, jnp.int32)])
    xp = jnp.pad(x, ((0, N_PAD - N), (0, 0)))
    eye8 = jnp.eye(HEADS, dtype=_f32)
    A1 = jnp.pad((a_src1[:, :, None] * eye8[:, None, :]).reshape(F1, HEADS),
                 ((0, 0), (0, 8)))
    A2 = jnp.pad((a_dst1[:, :, None] * eye8[:, None, :]).reshape(F1, HEADS),
                 ((0, 0), (0, 8)))
    R8 = jnp.asarray(np.kron(np.eye(8, dtype=np.float32),
                             np.ones((1, 16), np.float32)))
    AS2W = a_src2.reshape(NCLS, 1) * jnp.ones((1, 16), _f32)
    AD2W = a_dst2.reshape(NCLS, 1) * jnp.ones((1, 16), _f32)

    XS, AD = _k1(xp, W1, A1, A2)
    Z144 = jnp.zeros((N_PAD, F1 + 16), _f32)
    ACC1 = _passA(XS, AD, src, dst, Z144)
    TS2, TD2, DINV = _k2(ACC1, b1, W2, R8, AS2W, AD2W)
    Z32 = jnp.zeros((N_PAD, 32), _f32)
    ACC2 = _passCD(TS2, TD2, src, dst, Z32)
    G3, SELF = _k3(ACC2, b2, Wc, DINV)
    Z16 = jnp.zeros((N_PAD, 16), _f32)
    ACC3 = _passE(G3, src, dst, Z16)
    return _k4(ACC3, DINV, SELF, bc)[:N]


# speedup vs baseline: 68.3585x; 1.0006x over previous
import functools

import jax
import jax.numpy as jnp
import numpy as np
from jax import lax
from jax.experimental import pallas as pl
from jax.experimental.pallas import tpu as pltpu
from jax.experimental.pallas import tpu_sc as plsc

N = 10000
E = 320000
F_IN = 128
HEADS = 8
HID = 16
NCLS = 16
F1 = HEADS * HID

NC = 2
NS = 16
NW = NC * NS
N_PAD = 10016
E_PAD = 322560
EPW = E_PAD // NW
STRIPE = N_PAD // NS

_f32 = jnp.float32

_mesh = plsc.VectorSubcoreMesh(
    core_axis_name="c", subcore_axis_name="s", num_cores=NC, num_subcores=NS)


def _build_edge_pass(widths, idx_kinds, msg_w, compute_fn, ch, stage):
    n_t = len(widths)
    nchunk = EPW // ch
    assert EPW % ch == 0 and nchunk % 2 == 0 and ch % 16 == 0 and ch <= 128

    def body(*refs):
        t_hbm = refs[:n_t]
        src_hbm, dst_hbm, z_hbm, out_hbm = refs[n_t:n_t + 4]
        s = list(refs[n_t + 4:])
        srcg = s[0:2]
        dstg = s[2:4]
        dsts = s[4:6]
        gb = [s[6 + 2 * t:8 + 2 * t] for t in range(n_t)]
        o = 6 + 2 * n_t
        msg = s[o:o + 2]
        acc = s[o + 2]
        o += 3
        if stage:
            t_sh = s[o:o + n_t]
            o += n_t
        else:
            t_sh = t_hbm
        sem_ix = s[o:o + 2]
        sem_g = [s[o + 2 + 2 * t:o + 4 + 2 * t] for t in range(n_t)]
        sem_sc = s[o + 2 + 2 * n_t:o + 4 + 2 * n_t]
        gidx = [srcg if kind == "src" else dstg for kind in idx_kinds]

        cid = lax.axis_index("c")
        sid = lax.axis_index("s")
        wid = sid * NC + cid
        row0 = sid * STRIPE
        base = wid * EPW
        pltpu.sync_copy(z_hbm.at[pl.ds(row0, STRIPE)],
                        acc.at[pl.ds(row0, STRIPE)])
        if stage:
            for t in range(n_t):
                pltpu.sync_copy(t_hbm[t].at[pl.ds(row0, STRIPE)],
                                t_sh[t].at[pl.ds(row0, STRIPE)])
        plsc.subcore_barrier()

        def idx_slice(off):
            return pl.ds(pl.multiple_of(off, 8), ch)

        pltpu.sync_copy(src_hbm.at[idx_slice(base)], srcg[0])
        pltpu.sync_copy(dst_hbm.at[idx_slice(base)], dstg[0])
        for t in range(n_t):
            pltpu.async_copy(t_sh[t].at[gidx[t][0]], gb[t][0], sem_g[t][0])
        pltpu.async_copy(src_hbm.at[idx_slice(base + ch)], srcg[1], sem_ix[1])
        pltpu.async_copy(dst_hbm.at[idx_slice(base + ch)], dstg[1], sem_ix[1])

        def drain_scatter(b):
            pltpu.make_async_copy(msg[b], acc.at[dsts[b]], sem_sc[b]).wait()

        def half(k, b, nb, steady):
            if steady:
                pltpu.make_async_copy(src_hbm.at[idx_slice(base)], srcg[nb],
                                      sem_ix[nb]).wait()
                pltpu.make_async_copy(dst_hbm.at[idx_slice(base)], dstg[nb],
                                      sem_ix[nb]).wait()
                for t in range(n_t):
                    pltpu.async_copy(t_sh[t].at[gidx[t][nb]], gb[t][nb],
                                     sem_g[t][nb])

                @pl.when(k >= 2)
                def _():
                    drain_scatter(b)
            else:
                drain_scatter(b)
            for t in range(n_t):
                pltpu.make_async_copy(t_sh[t].at[gidx[t][b]], gb[t][b],
                                      sem_g[t][b]).wait()
            for q in range(ch // 16):
                dsts[b][pl.ds(16 * q, 16)] = dstg[b][pl.ds(16 * q, 16)]

            def edge(i, c2):
                compute_fn(i, [gb[t][b] for t in range(n_t)], msg[b])
                return c2

            lax.fori_loop(0, ch, edge, 0)
            pltpu.async_copy(msg[b], acc.at[dsts[b]], sem_sc[b], add=True)
            if steady:
                @pl.when(k + 2 <= nchunk - 1)
                def _():
                    off2 = base + (k + 2) * ch
                    pltpu.async_copy(src_hbm.at[idx_slice(off2)], srcg[b],
                                     sem_ix[b])
                    pltpu.async_copy(dst_hbm.at[idx_slice(off2)], dstg[b],
                                     sem_ix[b])

        def loop_body(j, c):
            half(2 * j, 0, 1, True)
            half(2 * j + 1, 1, 0, True)
            return c

        lax.fori_loop(0, (nchunk - 2) // 2, loop_body, 0)
        half(nchunk - 2, 0, 1, True)
        half(nchunk - 1, 1, 0, False)
        drain_scatter(0)
        drain_scatter(1)
        plsc.subcore_barrier()
        pltpu.sync_copy(acc.at[pl.ds(row0, STRIPE)],
                        out_hbm.at[cid, pl.ds(row0, STRIPE)])

    return body


def _edge_pass(widths, idx_kinds, msg_w, compute_fn, ch, stage=False):
    st = [pltpu.VMEM((ch,), jnp.int32) for _ in range(6)]
    for w in widths:
        st += [pltpu.VMEM((ch, w), _f32) for _ in range(2)]
    st += [pltpu.VMEM((ch, msg_w), _f32) for _ in range(2)]
    st += [pltpu.VMEM_SHARED((N_PAD, msg_w), _f32)]
    if stage:
        st += [pltpu.VMEM_SHARED((N_PAD, w), _f32) for w in widths]
    st += [pltpu.SemaphoreType.DMA for _ in range(4 + 2 * len(widths))]
    return functools.partial(
        pl.kernel,
        out_type=jax.ShapeDtypeStruct((NC, N_PAD, msg_w), _f32),
        mesh=_mesh,
        compiler_params=pltpu.CompilerParams(use_tc_tiling_on_sc=False),
        scratch_types=st,
    )(_build_edge_pass(widths, idx_kinds, msg_w, compute_fn, ch, stage))


def _passA_compute(i, gb, msg):
    xs_v, ad_v = gb
    a_s = xs_v[i, pl.ds(F1, 16)]
    a_d = ad_v[i, pl.ds(0, 16)]
    e = a_s + a_d
    w = jnp.exp(jnp.maximum(e, 0.2 * e))
    msg[i, pl.ds(F1, 16)] = w
    for h in range(HEADS):
        msg[i, pl.ds(h * HID, HID)] = xs_v[i, pl.ds(h * HID, HID)] * w[h]


_passA = _edge_pass([F1 + 16, 16], ["src", "dst"], F1 + 16, _passA_compute,
                    ch=48)


def _passCD_compute(i, gb, msg):
    ts_v, td_v = gb
    vx = ts_v[i, pl.ds(0, 16)]
    va = ts_v[i, pl.ds(16, 16)]
    vd = td_v[i, pl.ds(0, 16)]
    s = va + vd
    w = jnp.exp(jnp.maximum(s, 0.2 * s))
    msg[i, pl.ds(0, 16)] = vx * w
    msg[i, pl.ds(16, 16)] = w


_passCD = _edge_pass([32, 16], ["src", "dst"], 32, _passCD_compute,
                     ch=112, stage=True)


def _passE_compute(i, gb, msg):
    msg[i, pl.ds(0, 16)] = gb[0][i, pl.ds(0, 16)]


_passE = _edge_pass([16], ["src"], 16, _passE_compute, ch=112, stage=True)


def _k1_body(x_ref, w1_ref, a1_ref, a2_ref, xs_ref, ad_ref):
    xw = jnp.dot(x_ref[...], w1_ref[...], preferred_element_type=_f32)
    asp = jnp.dot(xw, a1_ref[...], preferred_element_type=_f32)
    xs_ref[...] = jnp.concatenate([xw, asp], axis=1)
    ad_ref[...] = jnp.dot(xw, a2_ref[...], preferred_element_type=_f32)


_k1 = pl.pallas_call(
    _k1_body,
    out_shape=[jax.ShapeDtypeStruct((N_PAD, F1 + 16), _f32),
               jax.ShapeDtypeStruct((N_PAD, 16), _f32)],
)


def _k2_body(acc_ref, b1_ref, w2_ref, r8_ref, as2w_ref, ad2w_ref,
             ts_ref, td_ref, dinv_ref):
    p = acc_ref[0] + acc_ref[1]
    den = jnp.dot(p[:, F1:F1 + 8], r8_ref[...], preferred_element_type=_f32)
    h1 = p[:, :F1] / (den + 1e-16) + b1_ref[...][None, :]
    h1 = jnp.where(h1 > 0, h1, jnp.exp(jnp.minimum(h1, 0.0)) - 1.0)
    x2 = jnp.dot(h1, w2_ref[...], preferred_element_type=_f32)
    as2 = jnp.dot(x2, as2w_ref[...], preferred_element_type=_f32)
    ts_ref[...] = jnp.concatenate([x2, as2], axis=1)
    td_ref[...] = jnp.dot(x2, ad2w_ref[...], preferred_element_type=_f32)
    deg = p[:, F1 + 8:F1 + 9]
    dinv_ref[...] = lax.rsqrt(1.0 + deg) * jnp.ones((1, 16), _f32)


_k2 = pl.pallas_call(
    _k2_body,
    out_shape=[jax.ShapeDtypeStruct((N_PAD, 32), _f32),
               jax.ShapeDtypeStruct((N_PAD, 16), _f32),
               jax.ShapeDtypeStruct((N_PAD, 16), _f32)],
)


def _k3_body(acc2_ref, b2_ref, wc_ref, dinv_ref, g3_ref, self_ref):
    p = acc2_ref[0] + acc2_ref[1]
    h2 = p[:, :16] / (p[:, 16:17] + 1e-16) + b2_ref[...][None, :]
    h3 = jnp.dot(h2, wc_ref[...], preferred_element_type=_f32)
    dv = dinv_ref[...]
    g3_ref[...] = h3 * dv
    self_ref[...] = h3 * dv * dv


_k3 = pl.pallas_call(
    _k3_body,
    out_shape=[jax.ShapeDtypeStruct((N_PAD, 16), _f32),
               jax.ShapeDtypeStruct((N_PAD, 16), _f32)],
)


def _k4_body(acc3_ref, dinv_ref, self_ref, bc_ref, out_ref):
    p = acc3_ref[0] + acc3_ref[1]
    out_ref[...] = dinv_ref[...] * p + self_ref[...] + bc_ref[...][None, :]


_k4 = pl.pallas_call(
    _k4_body,
    out_shape=jax.ShapeDtypeStruct((N_PAD, 16), _f32),
)


def kernel(x, edge_index, W1, a_src1, a_dst1, b1, W2, a_src2, a_dst2, b2,
           Wc, bc):
    src = edge_index[0].astype(jnp.int32)
    dst = edge_index[1].astype(jnp.int32)
    pad_e = E_PAD - E
    src = jnp.concatenate([src, jnp.full((pad_e,), N, jnp.int32)])
    dst = jnp.concatenate([dst, jnp.full((pad_e,), N, jnp.int32)])
    xp = jnp.pad(x, ((0, N_PAD - N), (0, 0)))
    eye8 = jnp.eye(HEADS, dtype=_f32)
    A1 = jnp.pad((a_src1[:, :, None] * eye8[:, None, :]).reshape(F1, HEADS),
                 ((0, 0), (0, 8)))
    A2 = jnp.pad((a_dst1[:, :, None] * eye8[:, None, :]).reshape(F1, HEADS),
                 ((0, 0), (0, 8)))
    R8 = jnp.asarray(np.kron(np.eye(8, dtype=np.float32),
                             np.ones((1, 16), np.float32)))
    AS2W = a_src2.reshape(NCLS, 1) * jnp.ones((1, 16), _f32)
    AD2W = a_dst2.reshape(NCLS, 1) * jnp.ones((1, 16), _f32)

    XS, AD = _k1(xp, W1, A1, A2)
    Z144 = jnp.zeros((N_PAD, F1 + 16), _f32)
    ACC1 = _passA(XS, AD, src, dst, Z144)
    TS2, TD2, DINV = _k2(ACC1, b1, W2, R8, AS2W, AD2W)
    Z32 = jnp.zeros((N_PAD, 32), _f32)
    ACC2 = _passCD(TS2, TD2, src, dst, Z32)
    G3, SELF = _k3(ACC2, b2, Wc, DINV)
    Z16 = jnp.zeros((N_PAD, 16), _f32)
    ACC3 = _passE(G3, src, dst, Z16)
    return _k4(ACC3, DINV, SELF, bc)[:N]
